# baseline (speedup 1.0000x reference)
import functools

import jax
import jax.numpy as jnp
from jax import lax
from jax.experimental import pallas as pl
from jax.experimental.pallas import tpu as pltpu
from jax.experimental.pallas import tpu_sc as plsc

NUM_FIELDS = 26
VOCAB = 100000
EMBED_DIM = 16
BATCH = 16384

NPAIR = NUM_FIELDS * EMBED_DIM
NC = 2
NS = 16
NW = NC * NS
PER_W = NPAIR // NW
QTR = BATCH // 4
LANES = 16


def _body(tab_hbm, xc_hbm, out_hbm, row_v, idx_v, out_v, row_sem, idx_sem, out_sem):
    cid = lax.axis_index("c")
    sid = lax.axis_index("s")
    wid = sid * NC + cid

    def gather_quarter(q):
        src = q * QTR
        dst = (q % 2) * QTR

        @plsc.parallel_loop(0, QTR, LANES, unroll=16)
        def _blk(off):
            out_v[pl.ds(dst + off, LANES)] = plsc.load_gather(
                row_v, [idx_v[pl.ds(src + off, LANES)]]
            )

    def start_out(p, q):
        return pltpu.async_copy(
            out_v.at[pl.ds((q % 2) * QTR, QTR)],
            out_hbm.at[p, pl.ds(q * QTR, QTR)],
            out_sem,
        )

    f_prev = jnp.int32(-1)
    pend_outs = []
    for j in range(PER_W):
        i = (j + sid) % PER_W
        p = wid * PER_W + i
        f = p // EMBED_DIM
        d = p % EMBED_DIM

        row_cp = pltpu.async_copy(tab_hbm.at[f, d, :], row_v, row_sem)

        @pl.when(f != f_prev)
        def _load_idx():
            pltpu.async_copy(xc_hbm.at[f, :], idx_v, idx_sem).wait()

        f_prev = f
        row_cp.wait()
        for cp in pend_outs:
            cp.wait()
        out_cps = [None, None, None, None]
        for q in range(4):
            if q >= 2:
                out_cps[q - 2].wait()
            gather_quarter(q)
            out_cps[q] = start_out(p, q)
        pend_outs = [out_cps[2], out_cps[3]]
    for cp in pend_outs:
        cp.wait()


@jax.jit
def _run(tab_t, xc_t):
    mesh = plsc.VectorSubcoreMesh(core_axis_name="c", subcore_axis_name="s")
    return pl.kernel(
        _body,
        out_type=jax.ShapeDtypeStruct((NPAIR, BATCH), jnp.float32),
        mesh=mesh,
        scratch_types=[
            pltpu.VMEM((VOCAB,), jnp.float32),
            pltpu.VMEM((BATCH,), jnp.int32),
            pltpu.VMEM((2 * QTR,), jnp.float32),
            pltpu.SemaphoreType.DMA,
            pltpu.SemaphoreType.DMA,
            pltpu.SemaphoreType.DMA,
        ],
        compiler_params=pltpu.CompilerParams(
            use_tc_tiling_on_sc=True, needs_layout_passes=False
        ),
    )(tab_t, xc_t)


def kernel(x_cat, tables):
    tab_t = jnp.transpose(tables, (0, 2, 1))
    xc_t = jnp.transpose(x_cat.astype(jnp.int32))
    out_t = _run(tab_t, xc_t)
    return jnp.transpose(out_t)


# speedup vs baseline: 1.0281x; 1.0281x over previous
import functools

import jax
import jax.numpy as jnp
from jax import lax
from jax.experimental import pallas as pl
from jax.experimental.pallas import tpu as pltpu
from jax.experimental.pallas import tpu_sc as plsc

NUM_FIELDS = 26
VOCAB = 100000
EMBED_DIM = 16
BATCH = 16384

NPAIR = NUM_FIELDS * EMBED_DIM
NC = 2
NS = 16
NW = NC * NS
PER_W = NPAIR // NW
QTR = BATCH // 4
LANES = 16


def _body(tab_hbm, xc_hbm, out_hbm, row_v, idx_v, out_v, row_sem, idx_sem, out_sem):
    cid = lax.axis_index("c")
    sid = lax.axis_index("s")
    wid = sid * NC + cid

    def gather_quarter(q):
        src = q * QTR
        dst = (q % 2) * QTR

        @plsc.parallel_loop(0, QTR, LANES, unroll=8)
        def _blk(off):
            out_v[pl.ds(dst + off, LANES)] = plsc.load_gather(
                row_v, [idx_v[pl.ds(src + off, LANES)]]
            )

    def start_out(p, q):
        return pltpu.async_copy(
            out_v.at[pl.ds((q % 2) * QTR, QTR)],
            out_hbm.at[p, pl.ds(q * QTR, QTR)],
            out_sem,
        )

    f_prev = jnp.int32(-1)
    pend_outs = []
    for j in range(PER_W):
        i = (j + sid) % PER_W
        p = wid * PER_W + i
        f = p // EMBED_DIM
        d = p % EMBED_DIM

        row_cp = pltpu.async_copy(tab_hbm.at[f, d, :], row_v, row_sem)

        @pl.when(f != f_prev)
        def _load_idx():
            pltpu.async_copy(xc_hbm.at[f, :], idx_v, idx_sem).wait()

        f_prev = f
        row_cp.wait()
        for cp in pend_outs:
            cp.wait()
        out_cps = [None, None, None, None]
        for q in range(4):
            if q >= 2:
                out_cps[q - 2].wait()
            gather_quarter(q)
            out_cps[q] = start_out(p, q)
        pend_outs = [out_cps[2], out_cps[3]]
    for cp in pend_outs:
        cp.wait()


@jax.jit
def _run(tab_t, xc_t):
    mesh = plsc.VectorSubcoreMesh(core_axis_name="c", subcore_axis_name="s")
    return pl.kernel(
        _body,
        out_type=jax.ShapeDtypeStruct((NPAIR, BATCH), jnp.float32),
        mesh=mesh,
        scratch_types=[
            pltpu.VMEM((VOCAB,), jnp.float32),
            pltpu.VMEM((BATCH,), jnp.int32),
            pltpu.VMEM((2 * QTR,), jnp.float32),
            pltpu.SemaphoreType.DMA,
            pltpu.SemaphoreType.DMA,
            pltpu.SemaphoreType.DMA,
        ],
        compiler_params=pltpu.CompilerParams(
            use_tc_tiling_on_sc=True, needs_layout_passes=False
        ),
    )(tab_t, xc_t)


def kernel(x_cat, tables):
    tab_t = jnp.transpose(tables, (0, 2, 1))
    xc_t = jnp.transpose(x_cat.astype(jnp.int32))
    out_t = _run(tab_t, xc_t)
    return jnp.transpose(out_t)


# speedup vs baseline: 1.0298x; 1.0017x over previous
import functools

import jax
import jax.numpy as jnp
from jax import lax
from jax.experimental import pallas as pl
from jax.experimental.pallas import tpu as pltpu
from jax.experimental.pallas import tpu_sc as plsc

NUM_FIELDS = 26
VOCAB = 100000
EMBED_DIM = 16
BATCH = 16384

NPAIR = NUM_FIELDS * EMBED_DIM
NC = 2
NS = 16
NW = NC * NS
PER_W = NPAIR // NW
QTR = BATCH // 4
LANES = 16


def _body(tab_hbm, xc_hbm, out_hbm, row_v, idx_v, out_v, row_sem, idx_sem, out_sem):
    cid = lax.axis_index("c")
    sid = lax.axis_index("s")
    wid = sid * NC + cid

    def gather_quarter(q):
        src = q * QTR
        dst = (q % 2) * QTR

        @plsc.parallel_loop(0, QTR, LANES, unroll=8)
        def _blk(off):
            out_v[pl.ds(dst + off, LANES)] = plsc.load_gather(
                row_v, [idx_v[pl.ds(src + off, LANES)]]
            )

    def start_out(p, q):
        return pltpu.async_copy(
            out_v.at[pl.ds((q % 2) * QTR, QTR)],
            out_hbm.at[p, pl.ds(q * QTR, QTR)],
            out_sem,
        )

    f_prev = jnp.int32(-1)
    pend_outs = []
    for j in range(PER_W):
        i = (j + sid) % PER_W
        p = wid * PER_W + i
        f = p // EMBED_DIM
        d = p % EMBED_DIM

        row_cp = pltpu.async_copy(tab_hbm.at[f, d, :], row_v, row_sem)

        @pl.when(f != f_prev)
        def _load_idx():
            pltpu.async_copy(xc_hbm.at[f, :], idx_v, idx_sem).wait()

        f_prev = f
        row_cp.wait()
        for cp in pend_outs:
            cp.wait()
        out_cps = [None, None, None, None]
        for q in range(4):
            if q >= 2:
                out_cps[q - 2].wait()
            gather_quarter(q)
            out_cps[q] = start_out(p, q)
        pend_outs = [out_cps[2], out_cps[3]]
    for cp in pend_outs:
        cp.wait()


@jax.jit
def _run(tab_t, xc_t):
    mesh = plsc.VectorSubcoreMesh(core_axis_name="c", subcore_axis_name="s")
    return pl.kernel(
        _body,
        out_type=jax.ShapeDtypeStruct((NPAIR, BATCH), jnp.float32),
        mesh=mesh,
        scratch_types=[
            pltpu.VMEM((VOCAB,), jnp.float32),
            pltpu.VMEM((BATCH,), jnp.int32),
            pltpu.VMEM((2 * QTR,), jnp.float32),
            pltpu.SemaphoreType.DMA,
            pltpu.SemaphoreType.DMA,
            pltpu.SemaphoreType.DMA,
        ],
        compiler_params=pltpu.CompilerParams(
            use_tc_tiling_on_sc=True,
            needs_layout_passes=False,
            skip_device_barrier=True,
        ),
    )(tab_t, xc_t)


def kernel(x_cat, tables):
    tab_t = jnp.transpose(tables, (0, 2, 1))
    xc_t = jnp.transpose(x_cat.astype(jnp.int32))
    out_t = _run(tab_t, xc_t)
    return jnp.transpose(out_t)


# speedup vs baseline: 1.0317x; 1.0018x over previous
import jax
import jax.numpy as jnp
from jax import lax
from jax.experimental import pallas as pl
from jax.experimental.pallas import tpu as pltpu
from jax.experimental.pallas import tpu_sc as plsc

NUM_FIELDS = 26
VOCAB = 100000
EMBED_DIM = 16
BATCH = 16384

NPAIR = NUM_FIELDS * EMBED_DIM
NC = 2
NS = 16
NW = NC * NS
PER_W = NPAIR // NW
QTR = BATCH // 4
LANES = 16


def _body(tab_hbm, xc_hbm, out_hbm, row_v, idx_v, out_v, row_sem, idx_sem, out_sem):
    cid = lax.axis_index("c")
    sid = lax.axis_index("s")
    wid = sid * NC + cid

    def gather_quarter(q):
        src = q * QTR
        dst = (q % 2) * QTR

        @plsc.parallel_loop(0, QTR, LANES, unroll=8)
        def _blk(off):
            out_v[pl.ds(dst + off, LANES)] = plsc.load_gather(
                row_v, [idx_v[pl.ds(src + off, LANES)]]
            )

    def start_out(p, q):
        return pltpu.async_copy(
            out_v.at[pl.ds((q % 2) * QTR, QTR)],
            out_hbm.at[p, pl.ds(q * QTR, QTR)],
            out_sem,
        )

    f_prev = jnp.int32(-1)
    pend_outs = []
    for j in range(PER_W):
        i = (j + sid) % PER_W
        p = wid * PER_W + i
        f = p // EMBED_DIM
        d = p % EMBED_DIM

        row_cp = pltpu.async_copy(tab_hbm.at[f, d, :], row_v, row_sem)

        @pl.when(f != f_prev)
        def _load_idx():
            pltpu.async_copy(xc_hbm.at[f, :], idx_v, idx_sem).wait()

        f_prev = f
        row_cp.wait()
        for cp in pend_outs:
            cp.wait()
        out_cps = [None, None, None, None]
        for q in range(4):
            if q >= 2:
                out_cps[q - 2].wait()
            gather_quarter(q)
            out_cps[q] = start_out(p, q)
        pend_outs = [out_cps[2], out_cps[3]]
    for cp in pend_outs:
        cp.wait()


@jax.jit
def _run(tab_t, xc_t):
    mesh = plsc.VectorSubcoreMesh(core_axis_name="c", subcore_axis_name="s")
    return pl.kernel(
        _body,
        out_type=jax.ShapeDtypeStruct((NPAIR, BATCH), jnp.float32),
        mesh=mesh,
        scratch_types=[
            pltpu.VMEM((VOCAB,), jnp.float32),
            pltpu.VMEM((BATCH,), jnp.int32),
            pltpu.VMEM((2 * QTR,), jnp.float32),
            pltpu.SemaphoreType.DMA,
            pltpu.SemaphoreType.DMA,
            pltpu.SemaphoreType.DMA,
        ],
        compiler_params=pltpu.CompilerParams(
            use_tc_tiling_on_sc=True, needs_layout_passes=False
        ),
    )(tab_t, xc_t)


def kernel(x_cat, tables):
    tab_t = jnp.transpose(tables, (0, 2, 1))
    xc_t = jnp.transpose(x_cat.astype(jnp.int32))
    out_t = _run(tab_t, xc_t)
    return jnp.transpose(out_t)
